# baseline (speedup 1.0000x reference)
import functools

import jax
import jax.numpy as jnp
from jax import lax
from jax.experimental import pallas as pl
from jax.experimental.pallas import tpu as pltpu
from jax.experimental.pallas import tpu_sc as plsc

N = 10000
E = 320000
D_HID = 16

NC = 2
NS = 16
NW = NC * NS
EW = E // NW
B = 1000
NB = EW // B
NBUF = 2
NG = NB // NBUF
NP = 10240
RPS = NP // NS

def _wid():
    return lax.axis_index("s") * NC + lax.axis_index("c")


@functools.cache
def _sc_kernels():
    mesh = plsc.VectorSubcoreMesh(
        core_axis_name="c", subcore_axis_name="s", num_cores=NC, num_subcores=NS
    )

    @functools.partial(
        pl.kernel,
        out_type=jax.ShapeDtypeStruct((NC, NP, D_HID), jnp.float32),
        mesh=mesh,
        compiler_params=pltpu.CompilerParams(use_tc_tiling_on_sc=False),
        scratch_types=[
            pltpu.VMEM((NB, B), jnp.int32),
            pltpu.VMEM((B, D_HID), jnp.float32),
            pltpu.VMEM_SHARED((NP, D_HID), jnp.float32),
            pltpu.SemaphoreType.DMA,
        ],
    )
    def sc_degree(dst_hbm, zeros_hbm, ones_hbm, out_hbm, didx_v, ones_v, acc_sh,
                  ssem):
        c = lax.axis_index("c")
        s = lax.axis_index("s")
        w = _wid()
        pltpu.sync_copy(zeros_hbm, acc_sh.at[pl.ds(s * RPS, RPS)])
        pltpu.sync_copy(ones_hbm, ones_v)
        pltpu.sync_copy(dst_hbm.at[w], didx_v)
        plsc.subcore_barrier()

        for j0 in range(NBUF):
            pltpu.async_copy(ones_v, acc_sh.at[didx_v.at[j0]], ssem, add=True)

        def body(j, carry):
            pltpu.make_async_copy(ones_v, acc_sh.at[didx_v.at[j]], ssem).wait()

            @pl.when(j + NBUF < NB)
            def _():
                pltpu.async_copy(
                    ones_v, acc_sh.at[didx_v.at[j + NBUF]], ssem, add=True
                )

            return carry

        lax.fori_loop(0, NB, body, 0)
        plsc.subcore_barrier()
        pltpu.sync_copy(
            acc_sh.at[pl.ds(s * RPS, RPS)], out_hbm.at[c, pl.ds(s * RPS, RPS)]
        )

    @functools.partial(
        pl.kernel,
        out_type=jax.ShapeDtypeStruct((NC, NP, D_HID), jnp.float32),
        mesh=mesh,
        compiler_params=pltpu.CompilerParams(use_tc_tiling_on_sc=False),
        scratch_types=[
            pltpu.VMEM((NB, B), jnp.int32),
            pltpu.VMEM((NB, B), jnp.int32),
            pltpu.VMEM((NBUF, B, D_HID), jnp.float32),
            pltpu.VMEM_SHARED((NP, D_HID), jnp.float32),
        ] + [pltpu.SemaphoreType.DMA] * (2 * NBUF),
    )
    def sc_prop(src_hbm, dst_hbm, hs_hbm, zeros_hbm, out_hbm,
                sidx_v, didx_v, msg_v, acc_sh, *sems):
        gsems = sems[:NBUF]
        ssems = sems[NBUF:]
        c = lax.axis_index("c")
        s = lax.axis_index("s")
        w = _wid()
        pltpu.sync_copy(src_hbm.at[w], sidx_v)
        pltpu.sync_copy(dst_hbm.at[w], didx_v)
        for b in range(NBUF):
            pltpu.async_copy(hs_hbm.at[sidx_v.at[b]], msg_v.at[b], gsems[b])
        pltpu.sync_copy(zeros_hbm, acc_sh.at[pl.ds(s * RPS, RPS)])
        plsc.subcore_barrier()

        def group(g, carry):
            for b in range(NBUF):
                j = g * NBUF + b
                pltpu.make_async_copy(
                    hs_hbm.at[sidx_v.at[j]], msg_v.at[b], gsems[b]
                ).wait()
                pltpu.async_copy(
                    msg_v.at[b], acc_sh.at[didx_v.at[j]], ssems[b], add=True
                )

            @pl.when(g + 1 < NG)
            def _():
                for b in range(NBUF):
                    j = (g + 1) * NBUF + b
                    pltpu.make_async_copy(
                        msg_v.at[b], acc_sh.at[didx_v.at[j - NBUF]], ssems[b]
                    ).wait()
                    pltpu.async_copy(
                        hs_hbm.at[sidx_v.at[j]], msg_v.at[b], gsems[b]
                    )

            return carry

        lax.fori_loop(0, NG, group, 0)
        for b in range(NBUF):
            pltpu.make_async_copy(
                msg_v.at[b], acc_sh.at[didx_v.at[(NG - 1) * NBUF + b]], ssems[b]
            ).wait()
        plsc.subcore_barrier()
        pltpu.sync_copy(
            acc_sh.at[pl.ds(s * RPS, RPS)], out_hbm.at[c, pl.ds(s * RPS, RPS)]
        )

    return sc_degree, sc_prop


_RB = 1000
_GRID = N // _RB


def _rows(block_cols):
    return pl.BlockSpec((_RB, block_cols), lambda i: (i, 0))


def _acc_spec():
    return pl.BlockSpec((NC, _RB, D_HID), lambda i: (0, i, 0))


def _full(shape):
    return pl.BlockSpec(shape, lambda i: (0,) * len(shape))


def _stage_a_body(x_ref, w1_ref, h0_ref):
    h0_ref[...] = jnp.dot(
        x_ref[...], w1_ref[...], preferred_element_type=jnp.float32
    )


def _stage_a(x, w1):
    return pl.pallas_call(
        _stage_a_body,
        grid=(_GRID,),
        in_specs=[_rows(128), _full((128, D_HID))],
        out_specs=_rows(D_HID),
        out_shape=jax.ShapeDtypeStruct((N, D_HID), jnp.float32),
    )(x, w1)


def _stage_b_body(h0_ref, deg_ref, hs0_ref, dinv_ref):
    deg = deg_ref[0, :, :] + deg_ref[1, :, :] + 1.0
    dinv = lax.rsqrt(deg)
    dinv_ref[...] = dinv
    hs0_ref[...] = dinv * h0_ref[...]


def _stage_b(h0, deg2):
    return pl.pallas_call(
        _stage_b_body,
        grid=(_GRID,),
        in_specs=[_rows(D_HID), _acc_spec()],
        out_specs=[_rows(D_HID), _rows(D_HID)],
        out_shape=[
            jax.ShapeDtypeStruct((N, D_HID), jnp.float32),
            jax.ShapeDtypeStruct((N, D_HID), jnp.float32),
        ],
    )(h0, deg2)


def _stage_c_body(acc_ref, hs0_ref, dinv_ref, w3_ref, b1_ref, hs1_ref):
    dinv = dinv_ref[...]
    p = dinv * (acc_ref[0, :, :] + acc_ref[1, :, :] + hs0_ref[...]) + b1_ref[...]
    h1 = jnp.maximum(p, 0.0)
    t1 = jnp.dot(h1, w3_ref[...], preferred_element_type=jnp.float32)
    hs1_ref[...] = dinv * t1


def _stage_c(acc, hs0, dinv, w3, b1):
    return pl.pallas_call(
        _stage_c_body,
        grid=(_GRID,),
        in_specs=[_acc_spec(), _rows(D_HID), _rows(D_HID),
                  _full((D_HID, D_HID)), _full((1, D_HID))],
        out_specs=_rows(D_HID),
        out_shape=jax.ShapeDtypeStruct((N, D_HID), jnp.float32),
    )(acc, hs0, dinv, w3, b1.reshape(1, D_HID))


def _stage_d_body(acc_ref, hs1_ref, dinv_ref, b3_ref, hs2_ref):
    dinv = dinv_ref[...]
    p = dinv * (acc_ref[0, :, :] + acc_ref[1, :, :] + hs1_ref[...]) + b3_ref[...]
    hs2_ref[...] = dinv * jnp.maximum(p, 0.0)


def _stage_d(acc, hs1, dinv, b3):
    return pl.pallas_call(
        _stage_d_body,
        grid=(_GRID,),
        in_specs=[_acc_spec(), _rows(D_HID), _rows(D_HID), _full((1, D_HID))],
        out_specs=_rows(D_HID),
        out_shape=jax.ShapeDtypeStruct((N, D_HID), jnp.float32),
    )(acc, hs1, dinv, b3.reshape(1, D_HID))


def _stage_e_body(acc_ref, hs2_ref, dinv_ref, w2_ref, b2_ref, out_ref):
    g = dinv_ref[...] * (acc_ref[0, :, :] + acc_ref[1, :, :] + hs2_ref[...])
    y = jnp.dot(g, w2_ref[...], preferred_element_type=jnp.float32) + b2_ref[...]
    m = jnp.max(y, axis=1, keepdims=True)
    lse = jnp.log(jnp.sum(jnp.exp(y - m), axis=1, keepdims=True)) + m
    out_ref[...] = y - lse


def _stage_e(acc, hs2, dinv, w2, b2):
    d_out = w2.shape[1]
    return pl.pallas_call(
        _stage_e_body,
        grid=(_GRID,),
        in_specs=[_acc_spec(), _rows(D_HID), _rows(D_HID),
                  _full((D_HID, d_out)), _full((1, d_out))],
        out_specs=_rows(d_out),
        out_shape=jax.ShapeDtypeStruct((N, d_out), jnp.float32),
    )(acc, hs2, dinv, w2, b2.reshape(1, d_out))


def kernel(x, edge_index, W1, b1, W3, b3, W2, b2):
    src = edge_index[0].astype(jnp.int32).reshape(NW, NB, B)
    dst = edge_index[1].astype(jnp.int32).reshape(NW, NB, B)
    zeros = jnp.zeros((RPS, D_HID), jnp.float32)
    ones = jnp.ones((B, D_HID), jnp.float32)

    sc_degree, sc_prop = _sc_kernels()
    h0 = _stage_a(x, W1)
    deg2 = sc_degree(dst, zeros, ones)
    hs0, dinv = _stage_b(h0, deg2)
    acc1 = sc_prop(src, dst, hs0, zeros)
    hs1 = _stage_c(acc1, hs0, dinv, W3, b1)
    acc2 = sc_prop(src, dst, hs1, zeros)
    hs2 = _stage_d(acc2, hs1, dinv, b3)
    acc3 = sc_prop(src, dst, hs2, zeros)
    return _stage_e(acc3, hs2, dinv, W2, b2)


# speedup vs baseline: 1.0451x; 1.0451x over previous
import functools

import jax
import jax.numpy as jnp
from jax import lax
from jax.experimental import pallas as pl
from jax.experimental.pallas import tpu as pltpu
from jax.experimental.pallas import tpu_sc as plsc

N = 10000
E = 320000
D_HID = 16

NC = 2
NS = 16
NW = NC * NS
EW = E // NW
B = 500
NB = EW // B
NBUF = 10
NG = NB // NBUF
NP = 10240
RPS = NP // NS

def _wid():
    return lax.axis_index("s") * NC + lax.axis_index("c")


@functools.cache
def _sc_kernels():
    mesh = plsc.VectorSubcoreMesh(
        core_axis_name="c", subcore_axis_name="s", num_cores=NC, num_subcores=NS
    )

    @functools.partial(
        pl.kernel,
        out_type=jax.ShapeDtypeStruct((NC, NP, D_HID), jnp.float32),
        mesh=mesh,
        compiler_params=pltpu.CompilerParams(use_tc_tiling_on_sc=False),
        scratch_types=[
            pltpu.VMEM((NB, B), jnp.int32),
            pltpu.VMEM((B, D_HID), jnp.float32),
            pltpu.VMEM_SHARED((NP, D_HID), jnp.float32),
            pltpu.SemaphoreType.DMA,
        ],
    )
    def sc_degree(dst_hbm, zeros_hbm, ones_hbm, out_hbm, didx_v, ones_v, acc_sh,
                  ssem):
        c = lax.axis_index("c")
        s = lax.axis_index("s")
        w = _wid()
        pltpu.sync_copy(zeros_hbm, acc_sh.at[pl.ds(s * RPS, RPS)])
        pltpu.sync_copy(ones_hbm, ones_v)
        pltpu.sync_copy(dst_hbm.at[w], didx_v)
        plsc.subcore_barrier()

        for j0 in range(NBUF):
            pltpu.async_copy(ones_v, acc_sh.at[didx_v.at[j0]], ssem, add=True)

        def body(j, carry):
            pltpu.make_async_copy(ones_v, acc_sh.at[didx_v.at[j]], ssem).wait()

            @pl.when(j + NBUF < NB)
            def _():
                pltpu.async_copy(
                    ones_v, acc_sh.at[didx_v.at[j + NBUF]], ssem, add=True
                )

            return carry

        lax.fori_loop(0, NB, body, 0)
        plsc.subcore_barrier()
        pltpu.sync_copy(
            acc_sh.at[pl.ds(s * RPS, RPS)], out_hbm.at[c, pl.ds(s * RPS, RPS)]
        )

    @functools.partial(
        pl.kernel,
        out_type=jax.ShapeDtypeStruct((NC, NP, D_HID), jnp.float32),
        mesh=mesh,
        compiler_params=pltpu.CompilerParams(use_tc_tiling_on_sc=False),
        scratch_types=[
            pltpu.VMEM((NB, B), jnp.int32),
            pltpu.VMEM((NB, B), jnp.int32),
            pltpu.VMEM((NBUF, B, D_HID), jnp.float32),
            pltpu.VMEM_SHARED((NP, D_HID), jnp.float32),
        ] + [pltpu.SemaphoreType.DMA] * (2 * NBUF),
    )
    def sc_prop(src_hbm, dst_hbm, hs_hbm, zeros_hbm, out_hbm,
                sidx_v, didx_v, msg_v, acc_sh, *sems):
        gsems = sems[:NBUF]
        ssems = sems[NBUF:]
        c = lax.axis_index("c")
        s = lax.axis_index("s")
        w = _wid()
        pltpu.sync_copy(src_hbm.at[w], sidx_v)
        pltpu.sync_copy(dst_hbm.at[w], didx_v)
        for b in range(NBUF):
            pltpu.async_copy(hs_hbm.at[sidx_v.at[b]], msg_v.at[b], gsems[b])
        pltpu.sync_copy(zeros_hbm, acc_sh.at[pl.ds(s * RPS, RPS)])
        plsc.subcore_barrier()

        def group(g, carry):
            for b in range(NBUF):
                j = g * NBUF + b
                pltpu.make_async_copy(
                    hs_hbm.at[sidx_v.at[j]], msg_v.at[b], gsems[b]
                ).wait()
                pltpu.async_copy(
                    msg_v.at[b], acc_sh.at[didx_v.at[j]], ssems[b], add=True
                )

            @pl.when(g + 1 < NG)
            def _():
                for b in range(NBUF):
                    j = (g + 1) * NBUF + b
                    pltpu.make_async_copy(
                        msg_v.at[b], acc_sh.at[didx_v.at[j - NBUF]], ssems[b]
                    ).wait()
                    pltpu.async_copy(
                        hs_hbm.at[sidx_v.at[j]], msg_v.at[b], gsems[b]
                    )

            return carry

        lax.fori_loop(0, NG, group, 0)
        for b in range(NBUF):
            pltpu.make_async_copy(
                msg_v.at[b], acc_sh.at[didx_v.at[(NG - 1) * NBUF + b]], ssems[b]
            ).wait()
        plsc.subcore_barrier()
        pltpu.sync_copy(
            acc_sh.at[pl.ds(s * RPS, RPS)], out_hbm.at[c, pl.ds(s * RPS, RPS)]
        )

    return sc_degree, sc_prop


_RB = 1000
_GRID = N // _RB


def _rows(block_cols):
    return pl.BlockSpec((_RB, block_cols), lambda i: (i, 0))


def _acc_spec():
    return pl.BlockSpec((NC, _RB, D_HID), lambda i: (0, i, 0))


def _full(shape):
    return pl.BlockSpec(shape, lambda i: (0,) * len(shape))


def _stage_a_body(x_ref, w1_ref, h0_ref):
    h0_ref[...] = jnp.dot(
        x_ref[...], w1_ref[...], preferred_element_type=jnp.float32
    )


def _stage_a(x, w1):
    return pl.pallas_call(
        _stage_a_body,
        grid=(_GRID,),
        in_specs=[_rows(128), _full((128, D_HID))],
        out_specs=_rows(D_HID),
        out_shape=jax.ShapeDtypeStruct((N, D_HID), jnp.float32),
    )(x, w1)


def _stage_b_body(h0_ref, deg_ref, hs0_ref, dinv_ref):
    deg = deg_ref[0, :, :] + deg_ref[1, :, :] + 1.0
    dinv = lax.rsqrt(deg)
    dinv_ref[...] = dinv
    hs0_ref[...] = dinv * h0_ref[...]


def _stage_b(h0, deg2):
    return pl.pallas_call(
        _stage_b_body,
        grid=(_GRID,),
        in_specs=[_rows(D_HID), _acc_spec()],
        out_specs=[_rows(D_HID), _rows(D_HID)],
        out_shape=[
            jax.ShapeDtypeStruct((N, D_HID), jnp.float32),
            jax.ShapeDtypeStruct((N, D_HID), jnp.float32),
        ],
    )(h0, deg2)


def _stage_c_body(acc_ref, hs0_ref, dinv_ref, w3_ref, b1_ref, hs1_ref):
    dinv = dinv_ref[...]
    p = dinv * (acc_ref[0, :, :] + acc_ref[1, :, :] + hs0_ref[...]) + b1_ref[...]
    h1 = jnp.maximum(p, 0.0)
    t1 = jnp.dot(h1, w3_ref[...], preferred_element_type=jnp.float32)
    hs1_ref[...] = dinv * t1


def _stage_c(acc, hs0, dinv, w3, b1):
    return pl.pallas_call(
        _stage_c_body,
        grid=(_GRID,),
        in_specs=[_acc_spec(), _rows(D_HID), _rows(D_HID),
                  _full((D_HID, D_HID)), _full((1, D_HID))],
        out_specs=_rows(D_HID),
        out_shape=jax.ShapeDtypeStruct((N, D_HID), jnp.float32),
    )(acc, hs0, dinv, w3, b1.reshape(1, D_HID))


def _stage_d_body(acc_ref, hs1_ref, dinv_ref, b3_ref, hs2_ref):
    dinv = dinv_ref[...]
    p = dinv * (acc_ref[0, :, :] + acc_ref[1, :, :] + hs1_ref[...]) + b3_ref[...]
    hs2_ref[...] = dinv * jnp.maximum(p, 0.0)


def _stage_d(acc, hs1, dinv, b3):
    return pl.pallas_call(
        _stage_d_body,
        grid=(_GRID,),
        in_specs=[_acc_spec(), _rows(D_HID), _rows(D_HID), _full((1, D_HID))],
        out_specs=_rows(D_HID),
        out_shape=jax.ShapeDtypeStruct((N, D_HID), jnp.float32),
    )(acc, hs1, dinv, b3.reshape(1, D_HID))


def _stage_e_body(acc_ref, hs2_ref, dinv_ref, w2_ref, b2_ref, out_ref):
    g = dinv_ref[...] * (acc_ref[0, :, :] + acc_ref[1, :, :] + hs2_ref[...])
    y = jnp.dot(g, w2_ref[...], preferred_element_type=jnp.float32) + b2_ref[...]
    m = jnp.max(y, axis=1, keepdims=True)
    lse = jnp.log(jnp.sum(jnp.exp(y - m), axis=1, keepdims=True)) + m
    out_ref[...] = y - lse


def _stage_e(acc, hs2, dinv, w2, b2):
    d_out = w2.shape[1]
    return pl.pallas_call(
        _stage_e_body,
        grid=(_GRID,),
        in_specs=[_acc_spec(), _rows(D_HID), _rows(D_HID),
                  _full((D_HID, d_out)), _full((1, d_out))],
        out_specs=_rows(d_out),
        out_shape=jax.ShapeDtypeStruct((N, d_out), jnp.float32),
    )(acc, hs2, dinv, w2, b2.reshape(1, d_out))


def kernel(x, edge_index, W1, b1, W3, b3, W2, b2):
    src = edge_index[0].astype(jnp.int32).reshape(NW, NB, B)
    dst = edge_index[1].astype(jnp.int32).reshape(NW, NB, B)
    zeros = jnp.zeros((RPS, D_HID), jnp.float32)
    ones = jnp.ones((B, D_HID), jnp.float32)

    sc_degree, sc_prop = _sc_kernels()
    h0 = _stage_a(x, W1)
    deg2 = sc_degree(dst, zeros, ones)
    hs0, dinv = _stage_b(h0, deg2)
    acc1 = sc_prop(src, dst, hs0, zeros)
    hs1 = _stage_c(acc1, hs0, dinv, W3, b1)
    acc2 = sc_prop(src, dst, hs1, zeros)
    hs2 = _stage_d(acc2, hs1, dinv, b3)
    acc3 = sc_prop(src, dst, hs2, zeros)
    return _stage_e(acc3, hs2, dinv, W2, b2)
